# Initial kernel scaffold; baseline (speedup 1.0000x reference)
import jax
import jax.numpy as jnp
from jax.experimental import pallas as pl


def kernel(x, edge_index, batch, dec_edge_index, params):
    raise NotImplementedError("write your pallas kernel here")


# speedup vs baseline: 21.2691x; 21.2691x over previous
import functools

import jax
import jax.numpy as jnp
from jax import lax
from jax.experimental import pallas as pl
from jax.experimental.pallas import tpu as pltpu
from jax.experimental.pallas import tpu_sc as plsc

N = 10000
E = 320000
D = 128
LATENT = 64
DEC_NODES = 196
NEG = 0.2

_F32 = jnp.float32
_NT = (((1,), (1,)), ((), ()))


_BR = 1000


def _mm_first_body(x_ref, w_ref, ap_ref, h_ref, a2_ref):
    h = jnp.dot(x_ref[...], w_ref[...], preferred_element_type=_F32)
    h_ref[...] = h
    a2_ref[...] = jnp.dot(h, ap_ref[...], preferred_element_type=_F32)


def _tc_first(x, w, ap):
    return pl.pallas_call(
        _mm_first_body,
        grid=(N // _BR,),
        in_specs=[
            pl.BlockSpec((_BR, D), lambda i: (i, 0)),
            pl.BlockSpec((D, D), lambda i: (0, 0)),
            pl.BlockSpec((D, 8), lambda i: (0, 0)),
        ],
        out_specs=[
            pl.BlockSpec((_BR, D), lambda i: (i, 0)),
            pl.BlockSpec((_BR, 8), lambda i: (i, 0)),
        ],
        out_shape=[
            jax.ShapeDtypeStruct((N, D), _F32),
            jax.ShapeDtypeStruct((N, 8), _F32),
        ],
    )(x, w, ap)


def _mm_comb_body(ph_ref, pw_ref, b_ref, w_ref, ap_ref, h_ref, a2_ref):
    num = ph_ref[0] + ph_ref[1]
    den = pw_ref[0, :, 0:1] + pw_ref[1, :, 0:1]
    hp = jnp.maximum(num / (den + 1e-16) + b_ref[...], 0.0)
    h = jnp.dot(hp, w_ref[...], preferred_element_type=_F32)
    h_ref[...] = h
    a2_ref[...] = jnp.dot(h, ap_ref[...], preferred_element_type=_F32)


def _tc_comb(ph, pw, b, w, ap):
    return pl.pallas_call(
        _mm_comb_body,
        grid=(N // _BR,),
        in_specs=[
            pl.BlockSpec((2, _BR, D), lambda i: (0, i, 0)),
            pl.BlockSpec((2, _BR, 16), lambda i: (0, i, 0)),
            pl.BlockSpec((1, D), lambda i: (0, 0)),
            pl.BlockSpec((D, D), lambda i: (0, 0)),
            pl.BlockSpec((D, 8), lambda i: (0, 0)),
        ],
        out_specs=[
            pl.BlockSpec((_BR, D), lambda i: (i, 0)),
            pl.BlockSpec((_BR, 8), lambda i: (i, 0)),
        ],
        out_shape=[
            jax.ShapeDtypeStruct((N, D), _F32),
            jax.ShapeDtypeStruct((N, 8), _F32),
        ],
    )(ph, pw, b, w, ap)


def _pool_body(ph_ref, pw_ref, b_ref, out_ref):
    i = pl.program_id(0)
    num = ph_ref[0] + ph_ref[1]
    den = pw_ref[0, :, 0:1] + pw_ref[1, :, 0:1]
    hp = jnp.maximum(num / (den + 1e-16) + b_ref[...], 0.0)

    @pl.when(i == 0)
    def _():
        out_ref[...] = jnp.zeros_like(out_ref)

    out_ref[...] += jnp.sum(hp, axis=0, keepdims=True)


def _tc_pool(ph, pw, b):
    return pl.pallas_call(
        _pool_body,
        grid=(N // _BR,),
        in_specs=[
            pl.BlockSpec((2, _BR, D), lambda i: (0, i, 0)),
            pl.BlockSpec((2, _BR, 16), lambda i: (0, i, 0)),
            pl.BlockSpec((1, D), lambda i: (0, 0)),
        ],
        out_specs=pl.BlockSpec((1, D), lambda i: (0, 0)),
        out_shape=jax.ShapeDtypeStruct((1, D), _F32),
    )(ph, pw, b)


def _fc_body(ps_ref, few_ref, feb_ref, fdw_ref, fdb_ref, out_ref):
    pooled = ps_ref[...] * (1.0 / N)
    z = jnp.dot(pooled, few_ref[...], preferred_element_type=_F32) + feb_ref[...]
    xf = jnp.dot(z, fdw_ref[...], preferred_element_type=_F32) + fdb_ref[...]
    out_ref[...] = jnp.maximum(xf, 0.0)


def _tc_fc(ps, few, feb, fdw, fdb):
    return pl.pallas_call(
        _fc_body,
        out_shape=jax.ShapeDtypeStruct((1, DEC_NODES * LATENT), _F32),
    )(ps, few, feb, fdw, fdb)


_DN = 256
_DE = 2432


def _dec_body(xd_ref, srcc_ref, dstr_ref, w_ref, a_ref, b_ref, out_ref):
    ohs = (srcc_ref[...] ==
           lax.broadcasted_iota(jnp.int32, (_DE, _DN), 1)).astype(_F32)
    ohdt = (dstr_ref[...] ==
            lax.broadcasted_iota(jnp.int32, (_DN, _DE), 0)).astype(_F32)
    mask = jnp.dot(ohdt, ohs, preferred_element_type=_F32)
    h = xd_ref[...]
    for l in range(4):
        hh = jnp.dot(h, w_ref[l], preferred_element_type=_F32)
        asrc = lax.dot_general(a_ref[l, 0:1, :], hh, _NT,
                               preferred_element_type=_F32)
        adst = lax.dot_general(hh, a_ref[l, 1:2, :], _NT,
                               preferred_element_type=_F32)
        e = adst + asrc
        e = jnp.where(e >= 0, e, e * NEG)
        em = jnp.where(mask > 0, e, -1e30)
        rmax = jnp.max(em, axis=1, keepdims=True)
        rmax = jnp.where(rmax > -1e29, rmax, 0.0)
        p = mask * jnp.exp(e - rmax)
        den = jnp.sum(p, axis=1, keepdims=True)
        coef = p / (den + 1e-16)
        h = jnp.dot(coef, hh, preferred_element_type=_F32) + b_ref[l]
        if l < 3:
            h = jnp.maximum(h, 0.0)
    out_ref[...] = h


def _tc_dec(xd, srcc, dstr, wst, ast, bst):
    return pl.pallas_call(
        _dec_body,
        out_shape=jax.ShapeDtypeStruct((_DN, D), _F32),
    )(xd, srcc, dstr, wst, ast, bst)


_NC = 2
_NS = 16
_NW = _NC * _NS
_EPW = E // _NW
_CH = 80
_NCHUNK = _EPW // _CH
_RPT = 624


def _make_sc_edge():
    mesh = plsc.VectorSubcoreMesh(core_axis_name="c", subcore_axis_name="s")

    @functools.partial(
        pl.kernel,
        out_type=(jax.ShapeDtypeStruct((_NC, N, D), _F32),
                  jax.ShapeDtypeStruct((_NC, N, 16), _F32)),
        mesh=mesh,
        scratch_types=[
            pltpu.VMEM((N,), _F32),
            pltpu.VMEM((N,), _F32),
            pltpu.VMEM((_CH, D), _F32),
            pltpu.VMEM((_CH, 16), _F32),
            pltpu.VMEM((_CH,), _F32),
            pltpu.VMEM((_CH,), jnp.int32),
            pltpu.VMEM((_CH,), jnp.int32),
            pltpu.VMEM_SHARED((N, D), _F32),
            pltpu.VMEM_SHARED((N, 16), _F32),
            pltpu.SemaphoreType.DMA,
        ],
        compiler_params=pltpu.CompilerParams(use_tc_tiling_on_sc=False,
                                             needs_layout_passes=False),
    )
    def sc_edge(src_hbm, dst_hbm, as_hbm, ad_hbm, h_hbm, ph_hbm, pw_hbm,
                asv, adv, rows, wst, wbuf, sidx, didx, accH, accW, sem):
        cid = lax.axis_index("c")
        sid = lax.axis_index("s")
        wid = sid * _NC + cid

        pltpu.sync_copy(as_hbm, asv)
        pltpu.sync_copy(ad_hbm, adv)
        ebase = wid * _EPW

        zero16 = jnp.zeros((16,), _F32)

        def zrow(r, carry):
            for jj in range(D // 16):
                rows[r, pl.ds(jj * 16, 16)] = zero16
            wst[r, pl.ds(0, 16)] = zero16
            return carry

        lax.fori_loop(0, _CH, zrow, 0)

        for roff2, ln in ((0, 80), (80, 80), (160, 80), (240, 80),
                          (320, 80), (400, 80), (480, 80), (560, 64)):
            roff = sid * _RPT + roff2
            pltpu.sync_copy(rows.at[pl.ds(0, ln)], accH.at[pl.ds(roff, ln)])
            pltpu.sync_copy(wst.at[pl.ds(0, ln)], accW.at[pl.ds(roff, ln)])

        @pl.when(sid < 2)
        def _():
            roff = _NS * _RPT + sid * 8
            pltpu.sync_copy(rows.at[pl.ds(0, 8)], accH.at[pl.ds(roff, 8)])
            pltpu.sync_copy(wst.at[pl.ds(0, 8)], accW.at[pl.ds(roff, 8)])

        plsc.subcore_barrier()

        lid0 = lax.iota(jnp.int32, 16)
        col0 = jnp.zeros((16,), jnp.int32)

        def chunk_body(c, carry):
            eoff = ebase + c * _CH
            pltpu.sync_copy(src_hbm.at[pl.ds(eoff, _CH)], sidx)
            pltpu.sync_copy(dst_hbm.at[pl.ds(eoff, _CH)], didx)
            for jj in range(_CH // 16):
                s16 = sidx[pl.ds(jj * 16, 16)]
                d16 = didx[pl.ds(jj * 16, 16)]
                av = plsc.load_gather(asv, [s16])
                dv = plsc.load_gather(adv, [d16])
                e = av + dv
                e = jnp.where(e >= 0, e, e * NEG)
                w = jnp.exp(e)
                wbuf[pl.ds(jj * 16, 16)] = w
                plsc.store_scatter(wst, [lid0 + jj * 16, col0], w)
            pltpu.async_copy(h_hbm.at[sidx], rows, sem).wait()

            def scale(r, carry2):
                wspl = plsc.load_gather(wbuf, [jnp.zeros((16,), jnp.int32) + r])
                for jj in range(D // 16):
                    rows[r, pl.ds(jj * 16, 16)] = (
                        rows[r, pl.ds(jj * 16, 16)] * wspl)
                return carry2

            lax.fori_loop(0, _CH, scale, 0)
            pltpu.sync_copy(rows, accH.at[didx], add=True)
            pltpu.sync_copy(wst, accW.at[didx], add=True)
            return carry

        lax.fori_loop(0, _NCHUNK, chunk_body, 0)

        plsc.subcore_barrier()
        roff = sid * _RPT
        pltpu.sync_copy(accH.at[pl.ds(roff, _RPT)],
                        ph_hbm.at[cid, pl.ds(roff, _RPT)])
        pltpu.sync_copy(accW.at[pl.ds(roff, _RPT)],
                        pw_hbm.at[cid, pl.ds(roff, _RPT)])

        @pl.when(sid < 2)
        def _():
            roff2 = _NS * _RPT + sid * 8
            pltpu.sync_copy(accH.at[pl.ds(roff2, 8)],
                            ph_hbm.at[cid, pl.ds(roff2, 8)])
            pltpu.sync_copy(accW.at[pl.ds(roff2, 8)],
                            pw_hbm.at[cid, pl.ds(roff2, 8)])

    return sc_edge


_SC_EDGE = _make_sc_edge()


def kernel(x, edge_index, batch, dec_edge_index, params):
    src = edge_index[0]
    dst = edge_index[1]

    enc = params["enc"]

    def apad(a):
        return jnp.pad(a.T, ((0, 0), (0, 6)))

    h, a2 = _tc_first(x, enc[0]["W"], apad(enc[0]["a"]))
    ph, pw = _SC_EDGE(src, dst, a2[:, 0], a2[:, 1], h)
    for li in (1, 2):
        p = enc[li]
        b_prev = enc[li - 1]["b"].reshape(1, D)
        h, a2 = _tc_comb(ph, pw, b_prev, p["W"], apad(p["a"]))
        ph, pw = _SC_EDGE(src, dst, a2[:, 0], a2[:, 1], h)

    pooled_sum = _tc_pool(ph, pw, enc[2]["b"].reshape(1, D))
    fe = params["fc_enc"]
    fd = params["fc_dec"]
    xflat = _tc_fc(pooled_sum, fe["W"], fe["b"].reshape(1, LATENT),
                   fd["W"], fd["b"].reshape(1, DEC_NODES * LATENT))

    xd0 = xflat.reshape(DEC_NODES, LATENT)
    xd0 = jnp.pad(xd0, ((0, _DN - DEC_NODES), (0, D - LATENT)))
    de = jnp.pad(dec_edge_index, ((0, 0), (0, _DE - dec_edge_index.shape[1])),
                 constant_values=_DN - 1)
    srcc = de[0].reshape(_DE, 1)
    dstr = de[1].reshape(1, _DE)
    dec = params["dec"]
    wst = jnp.stack([jnp.pad(dec[0]["W"], ((0, D - LATENT), (0, 0)))]
                    + [p["W"] for p in dec[1:]])
    ast = jnp.stack([p["a"] for p in dec])
    bst = jnp.stack([p["b"].reshape(1, D) for p in dec])
    out = _tc_dec(xd0, srcc, dstr, wst, ast, bst)
    return out[:DEC_NODES]


# speedup vs baseline: 32.3562x; 1.5213x over previous
import functools

import jax
import jax.numpy as jnp
from jax import lax
from jax.experimental import pallas as pl
from jax.experimental.pallas import tpu as pltpu
from jax.experimental.pallas import tpu_sc as plsc

N = 10000
E = 320000
D = 128
LATENT = 64
DEC_NODES = 196
NEG = 0.2

_F32 = jnp.float32
_NT = (((1,), (1,)), ((), ()))


_BR = 1000


def _mm_first_body(x_ref, w_ref, ap_ref, h_ref, a2_ref):
    h = jnp.dot(x_ref[...], w_ref[...], preferred_element_type=_F32)
    h_ref[...] = h
    a2_ref[...] = jnp.dot(h, ap_ref[...], preferred_element_type=_F32)


def _tc_first(x, w, ap):
    return pl.pallas_call(
        _mm_first_body,
        grid=(N // _BR,),
        in_specs=[
            pl.BlockSpec((_BR, D), lambda i: (i, 0)),
            pl.BlockSpec((D, D), lambda i: (0, 0)),
            pl.BlockSpec((D, 8), lambda i: (0, 0)),
        ],
        out_specs=[
            pl.BlockSpec((_BR, D), lambda i: (i, 0)),
            pl.BlockSpec((_BR, 8), lambda i: (i, 0)),
        ],
        out_shape=[
            jax.ShapeDtypeStruct((N, D), _F32),
            jax.ShapeDtypeStruct((N, 8), _F32),
        ],
    )(x, w, ap)


def _mm_comb_body(ph_ref, pw_ref, b_ref, w_ref, ap_ref, h_ref, a2_ref):
    num = ph_ref[0] + ph_ref[1]
    den = pw_ref[0] + pw_ref[1]
    hp = jnp.maximum(num / (den + 1e-16) + b_ref[...], 0.0)
    h = jnp.dot(hp, w_ref[...], preferred_element_type=_F32)
    h_ref[...] = h
    a2_ref[...] = jnp.dot(h, ap_ref[...], preferred_element_type=_F32)


def _tc_comb(ph, pw, b, w, ap):
    return pl.pallas_call(
        _mm_comb_body,
        grid=(N // _BR,),
        in_specs=[
            pl.BlockSpec((2, _BR, D), lambda i: (0, i, 0)),
            pl.BlockSpec((2, _BR, 1), lambda i: (0, i, 0)),
            pl.BlockSpec((1, D), lambda i: (0, 0)),
            pl.BlockSpec((D, D), lambda i: (0, 0)),
            pl.BlockSpec((D, 8), lambda i: (0, 0)),
        ],
        out_specs=[
            pl.BlockSpec((_BR, D), lambda i: (i, 0)),
            pl.BlockSpec((_BR, 8), lambda i: (i, 0)),
        ],
        out_shape=[
            jax.ShapeDtypeStruct((N, D), _F32),
            jax.ShapeDtypeStruct((N, 8), _F32),
        ],
    )(ph, pw, b, w, ap)


def _pool_body(ph_ref, pw_ref, b_ref, out_ref):
    i = pl.program_id(0)
    num = ph_ref[0] + ph_ref[1]
    den = pw_ref[0] + pw_ref[1]
    hp = jnp.maximum(num / (den + 1e-16) + b_ref[...], 0.0)

    @pl.when(i == 0)
    def _():
        out_ref[...] = jnp.zeros_like(out_ref)

    out_ref[...] += jnp.sum(hp, axis=0, keepdims=True)


def _tc_pool(ph, pw, b):
    return pl.pallas_call(
        _pool_body,
        grid=(N // _BR,),
        in_specs=[
            pl.BlockSpec((2, _BR, D), lambda i: (0, i, 0)),
            pl.BlockSpec((2, _BR, 1), lambda i: (0, i, 0)),
            pl.BlockSpec((1, D), lambda i: (0, 0)),
        ],
        out_specs=pl.BlockSpec((1, D), lambda i: (0, 0)),
        out_shape=jax.ShapeDtypeStruct((1, D), _F32),
    )(ph, pw, b)


def _fc_body(ps_ref, few_ref, feb_ref, fdw_ref, fdb_ref, out_ref):
    pooled = ps_ref[...] * (1.0 / N)
    z = jnp.dot(pooled, few_ref[...], preferred_element_type=_F32) + feb_ref[...]
    xf = jnp.dot(z, fdw_ref[...], preferred_element_type=_F32) + fdb_ref[...]
    out_ref[...] = jnp.maximum(xf, 0.0)


def _tc_fc(ps, few, feb, fdw, fdb):
    return pl.pallas_call(
        _fc_body,
        out_shape=jax.ShapeDtypeStruct((1, DEC_NODES * LATENT), _F32),
    )(ps, few, feb, fdw, fdb)


_DN = 256
_DE = 2432


def _dec_body(xd_ref, srcc_ref, dstr_ref, w_ref, a_ref, b_ref, out_ref):
    ohs = (srcc_ref[...] ==
           lax.broadcasted_iota(jnp.int32, (_DE, _DN), 1)).astype(_F32)
    ohdt = (dstr_ref[...] ==
            lax.broadcasted_iota(jnp.int32, (_DN, _DE), 0)).astype(_F32)
    mask = jnp.dot(ohdt, ohs, preferred_element_type=_F32)
    h = xd_ref[...]
    for l in range(4):
        hh = jnp.dot(h, w_ref[l], preferred_element_type=_F32)
        asrc = lax.dot_general(a_ref[l, 0:1, :], hh, _NT,
                               preferred_element_type=_F32)
        adst = lax.dot_general(hh, a_ref[l, 1:2, :], _NT,
                               preferred_element_type=_F32)
        e = adst + asrc
        e = jnp.where(e >= 0, e, e * NEG)
        em = jnp.where(mask > 0, e, -1e30)
        rmax = jnp.max(em, axis=1, keepdims=True)
        rmax = jnp.where(rmax > -1e29, rmax, 0.0)
        p = mask * jnp.exp(e - rmax)
        den = jnp.sum(p, axis=1, keepdims=True)
        coef = p / (den + 1e-16)
        h = jnp.dot(coef, hh, preferred_element_type=_F32) + b_ref[l]
        if l < 3:
            h = jnp.maximum(h, 0.0)
    out_ref[...] = h


def _tc_dec(xd, srcc, dstr, wst, ast, bst):
    return pl.pallas_call(
        _dec_body,
        out_shape=jax.ShapeDtypeStruct((_DN, D), _F32),
    )(xd, srcc, dstr, wst, ast, bst)


_NC = 2
_NS = 16
_NW = _NC * _NS
_EPW = E // _NW
_CH = 80
_NCHUNK = _EPW // _CH
_RPT = 624
_DW = 640
_DPT = _DW // _NS


def _make_sc_edge():
    mesh = plsc.VectorSubcoreMesh(core_axis_name="c", subcore_axis_name="s")

    @functools.partial(
        pl.kernel,
        out_type=(jax.ShapeDtypeStruct((_NC, N, D), _F32),
                  jax.ShapeDtypeStruct((_NC, _DW, 16), _F32)),
        mesh=mesh,
        scratch_types=[
            pltpu.VMEM((N,), _F32),
            pltpu.VMEM((N,), _F32),
            pltpu.VMEM((2, _CH, D), _F32),
            pltpu.VMEM((2, _CH, 16), _F32),
            pltpu.VMEM((2, _CH), _F32),
            pltpu.VMEM((2, _CH), jnp.int32),
            pltpu.VMEM((2, _CH), jnp.int32),
            pltpu.VMEM((2, _CH), jnp.int32),
            pltpu.VMEM_SHARED((N, D), _F32),
            pltpu.VMEM_SHARED((_DW, 16), _F32),
            pltpu.SemaphoreType.DMA,
            pltpu.SemaphoreType.DMA,
            pltpu.SemaphoreType.DMA,
            pltpu.SemaphoreType.DMA,
            pltpu.SemaphoreType.DMA,
            pltpu.SemaphoreType.DMA,
            pltpu.SemaphoreType.DMA,
            pltpu.SemaphoreType.DMA,
        ],
        compiler_params=pltpu.CompilerParams(use_tc_tiling_on_sc=False,
                                             needs_layout_passes=False),
    )
    def sc_edge(src_hbm, dst_hbm, as_hbm, ad_hbm, h_hbm, ph_hbm, pw_hbm,
                asv, adv, rows2, wst2, wbuf2, sidx2, didx2, dhi2, accH, accW,
                si0, si1, sg0, sg1, sh0, sh1, sw0, sw1):
        cid = lax.axis_index("c")
        sid = lax.axis_index("s")
        wid = sid * _NC + cid
        ebase = wid * _EPW

        bufs = [
            (rows2.at[0], wst2.at[0], wbuf2.at[0], sidx2.at[0],
             didx2.at[0], dhi2.at[0], si0, sg0, sh0, sw0),
            (rows2.at[1], wst2.at[1], wbuf2.at[1], sidx2.at[1],
             didx2.at[1], dhi2.at[1], si1, sg1, sh1, sw1),
        ]

        pltpu.sync_copy(as_hbm, asv)
        pltpu.sync_copy(ad_hbm, adv)

        zero16 = jnp.zeros((16,), _F32)
        rows0, wst0 = bufs[0][0], bufs[0][1]

        def zrow(r, carry):
            for jj in range(D // 16):
                rows0[r, pl.ds(jj * 16, 16)] = zero16
            wst0[r, pl.ds(0, 16)] = zero16
            return carry

        lax.fori_loop(0, _CH, zrow, 0)

        for roff2, ln in ((0, 80), (80, 80), (160, 80), (240, 80),
                          (320, 80), (400, 80), (480, 80), (560, 64)):
            roff = sid * _RPT + roff2
            pltpu.sync_copy(rows0.at[pl.ds(0, ln)], accH.at[pl.ds(roff, ln)])
        pltpu.sync_copy(wst0.at[pl.ds(0, _DPT)],
                        accW.at[pl.ds(sid * _DPT, _DPT)])

        @pl.when(sid < 2)
        def _():
            roff = _NS * _RPT + sid * 8
            pltpu.sync_copy(rows0.at[pl.ds(0, 8)], accH.at[pl.ds(roff, 8)])

        plsc.subcore_barrier()

        lid0 = lax.iota(jnp.int32, 16)

        def start_idx(c, B):
            _, _, _, sidx, didx, _, sem_i, _, _, _ = B
            eoff = ebase + c * _CH
            pltpu.async_copy(src_hbm.at[pl.ds(eoff, _CH)], sidx, sem_i)
            pltpu.async_copy(dst_hbm.at[pl.ds(eoff, _CH)], didx, sem_i)

        def wait_idx(B):
            _, _, _, sidx, didx, _, sem_i, _, _, _ = B
            pltpu.make_async_copy(src_hbm.at[pl.ds(0, _CH)], sidx, sem_i).wait()
            pltpu.make_async_copy(dst_hbm.at[pl.ds(0, _CH)], didx, sem_i).wait()

        def wait_scat(B):
            rows, wst, _, _, didx, dhi, _, _, sem_h, sem_w = B
            pltpu.make_async_copy(rows, accH.at[didx], sem_h).wait()
            pltpu.make_async_copy(wst, accW.at[dhi], sem_w).wait()

        def process(c, B, OB, guard_first, prefetch):
            rows, wst, wbuf, sidx, didx, dhi, _, sem_g, sem_h, sem_w = B
            if guard_first:
                @pl.when(c > 0)
                def _():
                    wait_scat(OB)
            else:
                wait_scat(OB)
            if prefetch:
                start_idx(c + 1, OB)
            wait_idx(B)
            gd = pltpu.async_copy(h_hbm.at[sidx], rows, sem_g)
            for jj in range(_CH // 16):
                s16 = sidx[pl.ds(jj * 16, 16)]
                d16 = didx[pl.ds(jj * 16, 16)]
                av = plsc.load_gather(asv, [s16])
                dv = plsc.load_gather(adv, [d16])
                e = av + dv
                e = jnp.where(e >= 0, e, e * NEG)
                w = jnp.exp(e)
                wbuf[pl.ds(jj * 16, 16)] = w
                dhi[pl.ds(jj * 16, 16)] = lax.shift_right_logical(d16, 4)
            gd.wait()

            @plsc.parallel_loop(0, _CH, unroll=2)
            def _(r):
                wspl = plsc.load_gather(wbuf, [jnp.zeros((16,), jnp.int32) + r])
                wst[r, pl.ds(0, 16)] = zero16
                for jj in range(D // 16):
                    rows[r, pl.ds(jj * 16, 16)] = (
                        rows[r, pl.ds(jj * 16, 16)] * wspl)

            for jj in range(_CH // 16):
                w16 = wbuf[pl.ds(jj * 16, 16)]
                d16 = didx[pl.ds(jj * 16, 16)]
                plsc.store_scatter(wst, [lid0 + jj * 16,
                                         jnp.bitwise_and(d16, 15)], w16)
            pltpu.async_copy(rows, accH.at[didx], sem_h, add=True)
            pltpu.async_copy(wst, accW.at[dhi], sem_w, add=True)

        start_idx(0, bufs[0])

        def outer(c2, carry):
            c = c2 * 2
            process(c, bufs[0], bufs[1], True, True)
            process(c + 1, bufs[1], bufs[0], False, True)
            return carry

        lax.fori_loop(0, _NCHUNK // 2, outer, 0)
        process(_NCHUNK - 1, bufs[0], bufs[1], False, False)
        wait_scat(bufs[0])

        plsc.subcore_barrier()
        roff = sid * _RPT
        pltpu.sync_copy(accH.at[pl.ds(roff, _RPT)],
                        ph_hbm.at[cid, pl.ds(roff, _RPT)])
        pltpu.sync_copy(accW.at[pl.ds(sid * _DPT, _DPT)],
                        pw_hbm.at[cid, pl.ds(sid * _DPT, _DPT)])

        @pl.when(sid < 2)
        def _():
            roff2 = _NS * _RPT + sid * 8
            pltpu.sync_copy(accH.at[pl.ds(roff2, 8)],
                            ph_hbm.at[cid, pl.ds(roff2, 8)])

    return sc_edge


_SC_EDGE = _make_sc_edge()


def kernel(x, edge_index, batch, dec_edge_index, params):
    src = edge_index[0]
    dst = edge_index[1]

    enc = params["enc"]

    def apad(a):
        return jnp.pad(a.T, ((0, 0), (0, 6)))

    def unpack_den(pw):
        return pw[:, :N // 16, :].reshape(_NC, N, 1)

    h, a2 = _tc_first(x, enc[0]["W"], apad(enc[0]["a"]))
    ph, pw = _SC_EDGE(src, dst, a2[:, 0], a2[:, 1], h)
    for li in (1, 2):
        p = enc[li]
        b_prev = enc[li - 1]["b"].reshape(1, D)
        h, a2 = _tc_comb(ph, unpack_den(pw), b_prev, p["W"], apad(p["a"]))
        ph, pw = _SC_EDGE(src, dst, a2[:, 0], a2[:, 1], h)

    pooled_sum = _tc_pool(ph, unpack_den(pw), enc[2]["b"].reshape(1, D))
    fe = params["fc_enc"]
    fd = params["fc_dec"]
    xflat = _tc_fc(pooled_sum, fe["W"], fe["b"].reshape(1, LATENT),
                   fd["W"], fd["b"].reshape(1, DEC_NODES * LATENT))

    xd0 = xflat.reshape(DEC_NODES, LATENT)
    xd0 = jnp.pad(xd0, ((0, _DN - DEC_NODES), (0, D - LATENT)))
    de = jnp.pad(dec_edge_index, ((0, 0), (0, _DE - dec_edge_index.shape[1])),
                 constant_values=_DN - 1)
    srcc = de[0].reshape(_DE, 1)
    dstr = de[1].reshape(1, _DE)
    dec = params["dec"]
    wst = jnp.stack([jnp.pad(dec[0]["W"], ((0, D - LATENT), (0, 0)))]
                    + [p["W"] for p in dec[1:]])
    ast = jnp.stack([p["a"] for p in dec])
    bst = jnp.stack([p["b"].reshape(1, D) for p in dec])
    out = _tc_dec(xd0, srcc, dstr, wst, ast, bst)
    return out[:DEC_NODES]
